# baseline (device time: 30310 ns/iter reference)
import jax
import jax.numpy as jnp
from jax import lax
from jax.experimental import pallas as pl
from jax.experimental.pallas import tpu as pltpu

B, S, H, Dh, Dr = 2, 256, 16, 64, 32
D = 1024
DC_LOCAL = 64
DC = 2 * DC_LOCAL
SCALE = (Dh + Dr) ** -0.5
BF = jnp.bfloat16
F32 = jnp.float32


def _dot(a, b):
    return jnp.dot(a, b, preferred_element_type=F32)


def _dot_nt(a, b):
    return lax.dot_general(
        a, b, (((1,), (1,)), ((), ())), preferred_element_type=F32
    )


def kernel(x, Wdkv, Wuk, Wuv, Wq, Wqr, Wkr, Wo):
    def body(
        x_ref, wdkrt_ref, wukuv_ref, wq_ref, wqr_ref, wo_ref,
        out_ref, c_comm, wuk_comm, wuv_comm, send_sems, recv_sems,
    ):
        my_x = lax.axis_index("x")
        my_y = lax.axis_index("y")
        my_z = lax.axis_index("z")
        partner = (1 - my_x, my_y, my_z)

        wdkrt = wdkrt_ref[...].astype(BF)
        xs = [x_ref[b].astype(BF) for b in range(B)]
        krs = []
        for b in range(B):
            ckr = _dot_nt(xs[b], wdkrt)
            c_comm[0, b] = ckr[:, :DC_LOCAL].astype(BF)
            krs.append(ckr[:, DC_LOCAL:].astype(BF))
        wuk_comm[0] = wukuv_ref[:DC_LOCAL].astype(BF)
        wuv_comm[0] = wukuv_ref[DC_LOCAL:].astype(BF)

        barrier = pltpu.get_barrier_semaphore()
        pl.semaphore_signal(
            barrier, inc=1, device_id=partner,
            device_id_type=pl.DeviceIdType.MESH,
        )
        pl.semaphore_wait(barrier, 1)

        rdmas = []
        for i, buf in enumerate((c_comm, wuk_comm, wuv_comm)):
            rdma = pltpu.make_async_remote_copy(
                src_ref=buf.at[0],
                dst_ref=buf.at[1],
                send_sem=send_sems.at[i],
                recv_sem=recv_sems.at[i],
                device_id=partner,
                device_id_type=pl.DeviceIdType.MESH,
            )
            rdma.start()
            rdmas.append(rdma)

        wq = (wq_ref[...] * SCALE).astype(BF)
        wqr = (wqr_ref[...] * SCALE).astype(BF)
        qs = [_dot(xs[b], wq).astype(BF) for b in range(B)]
        qrs = [_dot(xs[b], wqr).astype(BF) for b in range(B)]
        wo = wo_ref[...].astype(BF)

        for rdma in rdmas:
            rdma.wait()

        wuk_full = wuk_comm[...].reshape(DC, D)
        wuv_full = wuv_comm[...].reshape(DC, D)
        for b in range(B):
            ccat = jnp.concatenate([c_comm[0, b], c_comm[1, b]], axis=1)
            k_b = _dot(ccat, wuk_full).astype(BF)
            v_b = _dot(ccat, wuv_full).astype(BF)
            o_heads = []
            for h in range(H):
                hd = slice(h * Dh, (h + 1) * Dh)
                e = jnp.exp(
                    _dot_nt(qs[b][:, hd], k_b[:, hd])
                    + _dot_nt(qrs[b][:, h * Dr:(h + 1) * Dr], krs[b])
                )
                rs = jnp.sum(e, axis=1, keepdims=True)
                o_h = lax.dot_general(
                    e.astype(BF), v_b[:, hd], (((1,), (0,)), ((), ())),
                    preferred_element_type=F32,
                )
                o_heads.append((o_h / rs).astype(BF))
            out_ref[b] = _dot(jnp.concatenate(o_heads, axis=1), wo)

    wdkr_t = jnp.concatenate([Wdkv, Wkr], axis=1).T
    wukuv = jnp.concatenate([Wuk, Wuv], axis=0)
    return pl.pallas_call(
        body,
        out_shape=jax.ShapeDtypeStruct((B, S, D), jnp.float32),
        in_specs=[pl.BlockSpec(memory_space=pltpu.VMEM)] * 6,
        out_specs=pl.BlockSpec(memory_space=pltpu.VMEM),
        scratch_shapes=[
            pltpu.VMEM((2, B, S, DC_LOCAL), BF),
            pltpu.VMEM((2, DC_LOCAL, D), BF),
            pltpu.VMEM((2, DC_LOCAL, D), BF),
            pltpu.SemaphoreType.DMA((3,)),
            pltpu.SemaphoreType.DMA((3,)),
        ],
        compiler_params=pltpu.CompilerParams(collective_id=0),
    )(x, wdkr_t, wukuv, Wq, Wqr, Wo)


# device time: 25763 ns/iter; 1.1765x vs baseline; 1.1765x over previous
import jax
import jax.numpy as jnp
from jax import lax
from jax.experimental import pallas as pl
from jax.experimental.pallas import tpu as pltpu

B, S, H, Dh, Dr = 2, 256, 16, 64, 32
D = 1024
DC_LOCAL = 64
DC = 2 * DC_LOCAL
SCALE = (Dh + Dr) ** -0.5
BF = jnp.bfloat16
F32 = jnp.float32


def _dot(a, b):
    return jnp.dot(a, b, preferred_element_type=F32)


def _dot_nt(a, b):
    return lax.dot_general(
        a, b, (((1,), (1,)), ((), ())), preferred_element_type=F32
    )


def kernel(x, Wdkv, Wuk, Wuv, Wq, Wqr, Wkr, Wo):
    def body(
        x_ref, wdkrt_ref, wukuv_ref, wq_ref, wqr_ref, wo_ref,
        out_ref, c_comm, wuk_comm, wuv_comm, qcat, kcat, vcat,
        send_sems, recv_sems,
    ):
        my_x = lax.axis_index("x")
        my_y = lax.axis_index("y")
        my_z = lax.axis_index("z")
        partner = (1 - my_x, my_y, my_z)

        wdkrt = wdkrt_ref[...].astype(BF)
        xs = [x_ref[b].astype(BF) for b in range(B)]
        krs = []
        for b in range(B):
            ckr = _dot_nt(xs[b], wdkrt)
            c_comm[0, b] = ckr[:, :DC_LOCAL].astype(BF)
            krs.append(ckr[:, DC_LOCAL:].astype(BF))
        wuk_comm[0] = wukuv_ref[:DC_LOCAL].astype(BF)
        wuv_comm[0] = wukuv_ref[DC_LOCAL:].astype(BF)

        barrier = pltpu.get_barrier_semaphore()
        pl.semaphore_signal(
            barrier, inc=1, device_id=partner,
            device_id_type=pl.DeviceIdType.MESH,
        )
        pl.semaphore_wait(barrier, 1)

        rdmas = []
        for i, buf in enumerate((c_comm, wuk_comm, wuv_comm)):
            rdma = pltpu.make_async_remote_copy(
                src_ref=buf.at[0],
                dst_ref=buf.at[1],
                send_sem=send_sems.at[i],
                recv_sem=recv_sems.at[i],
                device_id=partner,
                device_id_type=pl.DeviceIdType.MESH,
            )
            rdma.start()
            rdmas.append(rdma)

        wq = (wq_ref[...] * SCALE).astype(BF)
        wqr = (wqr_ref[...] * SCALE).astype(BF)
        wo = wo_ref[...].astype(BF)
        z32 = jnp.zeros((S, Dr), BF)
        one32 = jnp.ones((S, Dr), BF)
        for b in range(B):
            q_f = _dot(xs[b], wq)
            qr_f = _dot(xs[b], wqr)
            for h in range(H):
                c0 = 128 * h
                qcat[b, :, c0:c0 + 64] = q_f[:, 64 * h:64 * h + 64].astype(BF)
                qcat[b, :, c0 + 64:c0 + 96] = (
                    qr_f[:, 32 * h:32 * h + 32].astype(BF)
                )
                qcat[b, :, c0 + 96:c0 + 128] = z32

        for rdma in rdmas:
            rdma.wait()

        wuk_full = wuk_comm[...].reshape(DC, D)
        wuv_full = wuv_comm[...].reshape(DC, D)
        for b in range(B):
            ccat = jnp.concatenate([c_comm[0, b], c_comm[1, b]], axis=1)
            k_f = _dot(ccat, wuk_full)
            v_f = _dot(ccat, wuv_full)
            for h in range(H):
                c0 = 128 * h
                kcat[b, :, c0:c0 + 64] = k_f[:, 64 * h:64 * h + 64].astype(BF)
                kcat[b, :, c0 + 64:c0 + 96] = krs[b]
                kcat[b, :, c0 + 96:c0 + 128] = z32
                vcat[b, :, c0:c0 + 64] = v_f[:, 64 * h:64 * h + 64].astype(BF)
                vcat[b, :, c0 + 64:c0 + 96] = one32
                vcat[b, :, c0 + 96:c0 + 128] = z32
            o_heads = []
            for h in range(H):
                c0 = 128 * h
                e = jnp.exp(
                    _dot_nt(qcat[b, :, c0:c0 + 128], kcat[b, :, c0:c0 + 128])
                )
                o128 = _dot(e.astype(BF), vcat[b, :, c0:c0 + 128])
                o_heads.append(
                    (o128[:, :Dh] / o128[:, Dh:Dh + 1]).astype(BF)
                )
            out_ref[b] = _dot(jnp.concatenate(o_heads, axis=1), wo)

    wdkr_t = jnp.concatenate([Wdkv, Wkr], axis=1).T
    wukuv = jnp.concatenate([Wuk, Wuv], axis=0)
    return pl.pallas_call(
        body,
        out_shape=jax.ShapeDtypeStruct((B, S, D), jnp.float32),
        in_specs=[pl.BlockSpec(memory_space=pltpu.VMEM)] * 6,
        out_specs=pl.BlockSpec(memory_space=pltpu.VMEM),
        scratch_shapes=[
            pltpu.VMEM((2, B, S, DC_LOCAL), BF),
            pltpu.VMEM((2, DC_LOCAL, D), BF),
            pltpu.VMEM((2, DC_LOCAL, D), BF),
            pltpu.VMEM((B, S, H * 128), BF),
            pltpu.VMEM((B, S, H * 128), BF),
            pltpu.VMEM((B, S, H * 128), BF),
            pltpu.SemaphoreType.DMA((3,)),
            pltpu.SemaphoreType.DMA((3,)),
        ],
        compiler_params=pltpu.CompilerParams(collective_id=0),
    )(x, wdkr_t, wukuv, Wq, Wqr, Wo)


# device time: 24916 ns/iter; 1.2165x vs baseline; 1.0340x over previous
import jax
import jax.numpy as jnp
from jax import lax
from jax.experimental import pallas as pl
from jax.experimental.pallas import tpu as pltpu

B, S, H, Dh, Dr = 2, 256, 16, 64, 32
D = 1024
DC_LOCAL = 64
DC = 2 * DC_LOCAL
SCALE = (Dh + Dr) ** -0.5
BF = jnp.bfloat16
F32 = jnp.float32


def _dot(a, b):
    return jnp.dot(a, b, preferred_element_type=F32)


def _dot_nt(a, b):
    return lax.dot_general(
        a, b, (((1,), (1,)), ((), ())), preferred_element_type=F32
    )


def kernel(x, Wdkv, Wuk, Wuv, Wq, Wqr, Wkr, Wo):
    def body(
        x_ref, wsmall_ref, wq_ref, wqr_ref, wo_ref,
        out_ref, c_comm, wuk_comm, wuv_comm, qcat, kcat, vcat,
        send_sems, recv_sems,
    ):
        my_x = lax.axis_index("x")
        my_y = lax.axis_index("y")
        my_z = lax.axis_index("z")
        partner = (1 - my_x, my_y, my_z)

        wdkrt = wsmall_ref[0:96].astype(BF)
        xs = [x_ref[b].astype(BF) for b in range(B)]
        krs = []
        for b in range(B):
            ckr = _dot_nt(xs[b], wdkrt)
            c_comm[0, b] = ckr[:, :DC_LOCAL].astype(BF)
            krs.append(ckr[:, DC_LOCAL:].astype(BF))
        wuk_comm[0] = wsmall_ref[96:160].astype(BF)
        wuv_comm[0] = wsmall_ref[160:224].astype(BF)

        barrier = pltpu.get_barrier_semaphore()
        pl.semaphore_signal(
            barrier, inc=1, device_id=partner,
            device_id_type=pl.DeviceIdType.MESH,
        )
        pl.semaphore_wait(barrier, 1)

        rdmas = []
        for i, buf in enumerate((c_comm, wuk_comm, wuv_comm)):
            rdma = pltpu.make_async_remote_copy(
                src_ref=buf.at[0],
                dst_ref=buf.at[1],
                send_sem=send_sems.at[i],
                recv_sem=recv_sems.at[i],
                device_id=partner,
                device_id_type=pl.DeviceIdType.MESH,
            )
            rdma.start()
            rdmas.append(rdma)

        wq = (wq_ref[...] * SCALE).astype(BF)
        wqr = (wqr_ref[...] * SCALE).astype(BF)
        wo = wo_ref[...].astype(BF)
        z32 = jnp.zeros((S, Dr), BF)
        one32 = jnp.ones((S, Dr), BF)
        for b in range(B):
            q_f = _dot(xs[b], wq)
            qr_f = _dot(xs[b], wqr)
            for h in range(H):
                c0 = 128 * h
                qcat[b, :, c0:c0 + 64] = q_f[:, 64 * h:64 * h + 64].astype(BF)
                qcat[b, :, c0 + 64:c0 + 96] = (
                    qr_f[:, 32 * h:32 * h + 32].astype(BF)
                )
                qcat[b, :, c0 + 96:c0 + 128] = z32

        for rdma in rdmas:
            rdma.wait()

        wuk_full = wuk_comm[...].reshape(DC, D)
        wuv_full = wuv_comm[...].reshape(DC, D)
        for b in range(B):
            ccat = jnp.concatenate([c_comm[0, b], c_comm[1, b]], axis=1)
            k_f = _dot(ccat, wuk_full)
            v_f = _dot(ccat, wuv_full)
            for h in range(H):
                c0 = 128 * h
                kcat[b, :, c0:c0 + 64] = k_f[:, 64 * h:64 * h + 64].astype(BF)
                kcat[b, :, c0 + 64:c0 + 96] = krs[b]
                kcat[b, :, c0 + 96:c0 + 128] = z32
                vcat[b, :, c0:c0 + 64] = v_f[:, 64 * h:64 * h + 64].astype(BF)
                vcat[b, :, c0 + 64:c0 + 96] = one32
                vcat[b, :, c0 + 96:c0 + 128] = z32
            o_heads = []
            for h in range(H):
                c0 = 128 * h
                e = jnp.exp(
                    _dot_nt(qcat[b, :, c0:c0 + 128], kcat[b, :, c0:c0 + 128])
                )
                o128 = _dot(e.astype(BF), vcat[b, :, c0:c0 + 128])
                o_heads.append(
                    (o128[:, :Dh] / o128[:, Dh:Dh + 1]).astype(BF)
                )
            out_ref[b] = _dot(jnp.concatenate(o_heads, axis=1), wo)

    wsmall = jnp.concatenate(
        [Wdkv.T, Wkr.T, Wuk, Wuv], axis=0
    )
    return pl.pallas_call(
        body,
        out_shape=jax.ShapeDtypeStruct((B, S, D), jnp.float32),
        in_specs=[pl.BlockSpec(memory_space=pltpu.VMEM)] * 5,
        out_specs=pl.BlockSpec(memory_space=pltpu.VMEM),
        scratch_shapes=[
            pltpu.VMEM((2, B, S, DC_LOCAL), BF),
            pltpu.VMEM((2, DC_LOCAL, D), BF),
            pltpu.VMEM((2, DC_LOCAL, D), BF),
            pltpu.VMEM((B, S, H * 128), BF),
            pltpu.VMEM((B, S, H * 128), BF),
            pltpu.VMEM((B, S, H * 128), BF),
            pltpu.SemaphoreType.DMA((3,)),
            pltpu.SemaphoreType.DMA((3,)),
        ],
        compiler_params=pltpu.CompilerParams(collective_id=0),
    )(x, wsmall, Wq, Wqr, Wo)


# device time: 20285 ns/iter; 1.4942x vs baseline; 1.2283x over previous
import jax
import jax.numpy as jnp
from jax import lax
from jax.experimental import pallas as pl
from jax.experimental.pallas import tpu as pltpu

B, S, H, Dh, Dr = 2, 256, 16, 64, 32
D = 1024
DC_LOCAL = 64
DC = 2 * DC_LOCAL
SCALE = (Dh + Dr) ** -0.5
BF = jnp.bfloat16
F32 = jnp.float32


def _dot(a, b):
    return jnp.dot(a, b, preferred_element_type=F32)


def _dot_nt(a, b):
    return lax.dot_general(
        a, b, (((1,), (1,)), ((), ())), preferred_element_type=F32
    )


def kernel(x, Wdkv, Wuk, Wuv, Wq, Wqr, Wkr, Wo):
    def body(
        x_h, wsmall_h, wq_h, wqr_h, wo_h,
        out_ref, x_ref, wsmall_ref, wq_ref, wqr_ref, wo_ref,
        c_comm, wuk_comm, wuv_comm, qcat, kcat, vcat,
        load_sems, send_sems, recv_sems,
    ):
        my_x = lax.axis_index("x")
        my_y = lax.axis_index("y")
        my_z = lax.axis_index("z")
        partner = (1 - my_x, my_y, my_z)

        pairs = [
            (wsmall_h, wsmall_ref), (x_h, x_ref),
            (wq_h, wq_ref), (wqr_h, wqr_ref), (wo_h, wo_ref),
        ]
        cps = []
        for i, (h, v) in enumerate(pairs):
            cp = pltpu.make_async_copy(h, v, load_sems.at[i])
            cp.start()
            cps.append(cp)
        cp_wsmall, cp_x, cp_wq, cp_wqr, cp_wo = cps

        cp_wsmall.wait()
        cp_x.wait()
        wdkrt = wsmall_ref[0:96].astype(BF)
        xs = [x_ref[b].astype(BF) for b in range(B)]
        krs = []
        for b in range(B):
            ckr = _dot_nt(xs[b], wdkrt)
            c_comm[0, b] = ckr[:, :DC_LOCAL].astype(BF)
            krs.append(ckr[:, DC_LOCAL:].astype(BF))
        wuk_comm[0] = wsmall_ref[96:160].astype(BF)
        wuv_comm[0] = wsmall_ref[160:224].astype(BF)

        barrier = pltpu.get_barrier_semaphore()
        pl.semaphore_signal(
            barrier, inc=1, device_id=partner,
            device_id_type=pl.DeviceIdType.MESH,
        )
        pl.semaphore_wait(barrier, 1)

        rdmas = []
        for i, buf in enumerate((c_comm, wuk_comm, wuv_comm)):
            rdma = pltpu.make_async_remote_copy(
                src_ref=buf.at[0],
                dst_ref=buf.at[1],
                send_sem=send_sems.at[i],
                recv_sem=recv_sems.at[i],
                device_id=partner,
                device_id_type=pl.DeviceIdType.MESH,
            )
            rdma.start()
            rdmas.append(rdma)

        cp_wq.wait()
        wq = (wq_ref[...] * SCALE).astype(BF)
        cp_wqr.wait()
        wqr = (wqr_ref[...] * SCALE).astype(BF)
        cp_wo.wait()
        wo = wo_ref[...].astype(BF)
        z32 = jnp.zeros((S, Dr), BF)
        one32 = jnp.ones((S, Dr), BF)
        for b in range(B):
            q_f = _dot(xs[b], wq)
            qr_f = _dot(xs[b], wqr)
            for h in range(H):
                c0 = 128 * h
                qcat[b, :, c0:c0 + 64] = q_f[:, 64 * h:64 * h + 64].astype(BF)
                qcat[b, :, c0 + 64:c0 + 96] = (
                    qr_f[:, 32 * h:32 * h + 32].astype(BF)
                )
                qcat[b, :, c0 + 96:c0 + 128] = z32

        for rdma in rdmas:
            rdma.wait()

        wuk_full = wuk_comm[...].reshape(DC, D)
        wuv_full = wuv_comm[...].reshape(DC, D)
        for b in range(B):
            ccat = jnp.concatenate([c_comm[0, b], c_comm[1, b]], axis=1)
            k_f = _dot(ccat, wuk_full)
            v_f = _dot(ccat, wuv_full)
            for h in range(H):
                c0 = 128 * h
                kcat[b, :, c0:c0 + 64] = k_f[:, 64 * h:64 * h + 64].astype(BF)
                kcat[b, :, c0 + 64:c0 + 96] = krs[b]
                kcat[b, :, c0 + 96:c0 + 128] = z32
                vcat[b, :, c0:c0 + 64] = v_f[:, 64 * h:64 * h + 64].astype(BF)
                vcat[b, :, c0 + 64:c0 + 96] = one32
                vcat[b, :, c0 + 96:c0 + 128] = z32
            o_heads = []
            for h in range(H):
                c0 = 128 * h
                e = jnp.exp(
                    _dot_nt(qcat[b, :, c0:c0 + 128], kcat[b, :, c0:c0 + 128])
                )
                o128 = _dot(e.astype(BF), vcat[b, :, c0:c0 + 128])
                o_heads.append(
                    (o128[:, :Dh] / o128[:, Dh:Dh + 1]).astype(BF)
                )
            out_ref[b] = _dot(jnp.concatenate(o_heads, axis=1), wo)

    wsmall = jnp.concatenate(
        [Wdkv.T, Wkr.T, Wuk, Wuv], axis=0
    )
    hbm = lambda v: pltpu.with_memory_space_constraint(
        v, pltpu.MemorySpace.HBM
    )
    return pl.pallas_call(
        body,
        out_shape=jax.ShapeDtypeStruct((B, S, D), jnp.float32),
        in_specs=[pl.BlockSpec(memory_space=pltpu.MemorySpace.HBM)] * 5,
        out_specs=pl.BlockSpec(memory_space=pltpu.VMEM),
        scratch_shapes=[
            pltpu.VMEM((B, S, D), F32),
            pltpu.VMEM((224, D), F32),
            pltpu.VMEM((D, D), F32),
            pltpu.VMEM((D, H * Dr), F32),
            pltpu.VMEM((D, D), F32),
            pltpu.VMEM((2, B, S, DC_LOCAL), BF),
            pltpu.VMEM((2, DC_LOCAL, D), BF),
            pltpu.VMEM((2, DC_LOCAL, D), BF),
            pltpu.VMEM((B, S, H * 128), BF),
            pltpu.VMEM((B, S, H * 128), BF),
            pltpu.VMEM((B, S, H * 128), BF),
            pltpu.SemaphoreType.DMA((5,)),
            pltpu.SemaphoreType.DMA((3,)),
            pltpu.SemaphoreType.DMA((3,)),
        ],
        compiler_params=pltpu.CompilerParams(collective_id=0),
    )(hbm(x), hbm(wsmall), hbm(Wq), hbm(Wqr), hbm(Wo))


# device time: 18092 ns/iter; 1.6753x vs baseline; 1.1212x over previous
import jax
import jax.numpy as jnp
from jax import lax
from jax.experimental import pallas as pl
from jax.experimental.pallas import tpu as pltpu

B, S, H, Dh, Dr = 2, 256, 16, 64, 32
D = 1024
DC_LOCAL = 64
DC = 2 * DC_LOCAL
SCALE = (Dh + Dr) ** -0.5
BF = jnp.bfloat16
F32 = jnp.float32


def _dot(a, b):
    return jnp.dot(a, b, preferred_element_type=F32)


def _dot_nt(a, b):
    return lax.dot_general(
        a, b, (((1,), (1,)), ((), ())), preferred_element_type=F32
    )


def kernel(x, Wdkv, Wuk, Wuv, Wq, Wqr, Wkr, Wo):
    def body(
        x_h, wdkvt_h, wkrt_h, wuk_h, wuv_h, wq_h, wqr_h, wo_h,
        out_ref, x_ref, wsmall_ref, wq_ref, wqr_ref, wo_ref,
        c_comm, wuk_comm, wuv_comm, qcat, kcat, vcat,
        load_sems, send_sems, recv_sems,
    ):
        my_x = lax.axis_index("x")
        my_y = lax.axis_index("y")
        my_z = lax.axis_index("z")
        partner = (1 - my_x, my_y, my_z)

        pairs = [
            (wdkvt_h, wsmall_ref.at[0:64]),
            (wkrt_h, wsmall_ref.at[64:96]),
            (wuk_h, wsmall_ref.at[96:160]),
            (wuv_h, wsmall_ref.at[160:224]),
            (x_h, x_ref),
            (wq_h, wq_ref), (wqr_h, wqr_ref), (wo_h, wo_ref),
        ]
        cps = []
        for i, (h, v) in enumerate(pairs):
            cp = pltpu.make_async_copy(h, v, load_sems.at[i])
            cp.start()
            cps.append(cp)
        cp_wq, cp_wqr, cp_wo = cps[5], cps[6], cps[7]

        for cp in cps[:5]:
            cp.wait()
        wdkrt = wsmall_ref[0:96].astype(BF)
        xs = [x_ref[b].astype(BF) for b in range(B)]
        krs = []
        for b in range(B):
            ckr = _dot_nt(xs[b], wdkrt)
            c_comm[0, b] = ckr[:, :DC_LOCAL].astype(BF)
            krs.append(ckr[:, DC_LOCAL:].astype(BF))
        wuk_comm[0] = wsmall_ref[96:160].astype(BF)
        wuv_comm[0] = wsmall_ref[160:224].astype(BF)

        barrier = pltpu.get_barrier_semaphore()
        pl.semaphore_signal(
            barrier, inc=1, device_id=partner,
            device_id_type=pl.DeviceIdType.MESH,
        )
        pl.semaphore_wait(barrier, 1)

        rdmas = []
        for i, buf in enumerate((c_comm, wuk_comm, wuv_comm)):
            rdma = pltpu.make_async_remote_copy(
                src_ref=buf.at[0],
                dst_ref=buf.at[1],
                send_sem=send_sems.at[i],
                recv_sem=recv_sems.at[i],
                device_id=partner,
                device_id_type=pl.DeviceIdType.MESH,
            )
            rdma.start()
            rdmas.append(rdma)

        cp_wq.wait()
        wq = (wq_ref[...] * SCALE).astype(BF)
        cp_wqr.wait()
        wqr = (wqr_ref[...] * SCALE).astype(BF)
        cp_wo.wait()
        wo = wo_ref[...].astype(BF)
        z32 = jnp.zeros((S, Dr), BF)
        one32 = jnp.ones((S, Dr), BF)
        for b in range(B):
            q_f = _dot(xs[b], wq)
            qr_f = _dot(xs[b], wqr)
            for h in range(H):
                c0 = 128 * h
                qcat[b, :, c0:c0 + 64] = q_f[:, 64 * h:64 * h + 64].astype(BF)
                qcat[b, :, c0 + 64:c0 + 96] = (
                    qr_f[:, 32 * h:32 * h + 32].astype(BF)
                )
                qcat[b, :, c0 + 96:c0 + 128] = z32

        for rdma in rdmas:
            rdma.wait()

        wuk_full = wuk_comm[...].reshape(DC, D)
        wuv_full = wuv_comm[...].reshape(DC, D)
        for b in range(B):
            ccat = jnp.concatenate([c_comm[0, b], c_comm[1, b]], axis=1)
            k_f = _dot(ccat, wuk_full)
            v_f = _dot(ccat, wuv_full)
            for h in range(H):
                c0 = 128 * h
                kcat[b, :, c0:c0 + 64] = k_f[:, 64 * h:64 * h + 64].astype(BF)
                kcat[b, :, c0 + 64:c0 + 96] = krs[b]
                kcat[b, :, c0 + 96:c0 + 128] = z32
                vcat[b, :, c0:c0 + 64] = v_f[:, 64 * h:64 * h + 64].astype(BF)
                vcat[b, :, c0 + 64:c0 + 96] = one32
                vcat[b, :, c0 + 96:c0 + 128] = z32
            o_heads = []
            for h in range(H):
                c0 = 128 * h
                e = jnp.exp(
                    _dot_nt(qcat[b, :, c0:c0 + 128], kcat[b, :, c0:c0 + 128])
                )
                o128 = _dot(e.astype(BF), vcat[b, :, c0:c0 + 128])
                o_heads.append(
                    (o128[:, :Dh] / o128[:, Dh:Dh + 1]).astype(BF)
                )
            out_ref[b] = _dot(jnp.concatenate(o_heads, axis=1), wo)

    hbm = lambda v: pltpu.with_memory_space_constraint(
        v, pltpu.MemorySpace.HBM
    )
    return pl.pallas_call(
        body,
        out_shape=jax.ShapeDtypeStruct((B, S, D), jnp.float32),
        in_specs=[pl.BlockSpec(memory_space=pltpu.MemorySpace.HBM)] * 8,
        out_specs=pl.BlockSpec(memory_space=pltpu.VMEM),
        scratch_shapes=[
            pltpu.VMEM((B, S, D), F32),
            pltpu.VMEM((224, D), F32),
            pltpu.VMEM((D, D), F32),
            pltpu.VMEM((D, H * Dr), F32),
            pltpu.VMEM((D, D), F32),
            pltpu.VMEM((2, B, S, DC_LOCAL), BF),
            pltpu.VMEM((2, DC_LOCAL, D), BF),
            pltpu.VMEM((2, DC_LOCAL, D), BF),
            pltpu.VMEM((B, S, H * 128), BF),
            pltpu.VMEM((B, S, H * 128), BF),
            pltpu.VMEM((B, S, H * 128), BF),
            pltpu.SemaphoreType.DMA((8,)),
            pltpu.SemaphoreType.DMA((3,)),
            pltpu.SemaphoreType.DMA((3,)),
        ],
        compiler_params=pltpu.CompilerParams(collective_id=0),
    )(
        hbm(x), hbm(Wdkv.T), hbm(Wkr.T), hbm(Wuk), hbm(Wuv),
        hbm(Wq), hbm(Wqr), hbm(Wo),
    )


# device time: 17996 ns/iter; 1.6843x vs baseline; 1.0053x over previous
import jax
import jax.numpy as jnp
from jax import lax
from jax.experimental import pallas as pl
from jax.experimental.pallas import tpu as pltpu

B, S, H, Dh, Dr = 2, 256, 16, 64, 32
D = 1024
DC_LOCAL = 64
DC = 2 * DC_LOCAL
SCALE = (Dh + Dr) ** -0.5
BF = jnp.bfloat16
F32 = jnp.float32


def _dot(a, b):
    return jnp.dot(a, b, preferred_element_type=F32)


def _dot_nt(a, b):
    return lax.dot_general(
        a, b, (((1,), (1,)), ((), ())), preferred_element_type=F32
    )


def kernel(x, Wdkv, Wuk, Wuv, Wq, Wqr, Wkr, Wo):
    def body(
        x_h, wdkvt_h, wkrt_h, wuk_h, wuv_h, wq_h, wqr_h, wo_h,
        out_ref, x_ref, wsmall_ref, wq_ref, wqr_ref, wo_ref,
        c_comm, wuk_comm, wuv_comm, qcat, kcat, vcat,
        load_sems, send_sems, recv_sems,
    ):
        my_x = lax.axis_index("x")
        my_y = lax.axis_index("y")
        my_z = lax.axis_index("z")
        partner = (1 - my_x, my_y, my_z)

        pairs = [
            (wdkvt_h, wsmall_ref.at[0:64]),
            (wkrt_h, wsmall_ref.at[64:96]),
            (x_h.at[0], x_ref.at[0]),
            (x_h.at[1], x_ref.at[1]),
            (wuk_h, wsmall_ref.at[96:160]),
            (wuv_h, wsmall_ref.at[160:224]),
            (wq_h, wq_ref), (wqr_h, wqr_ref), (wo_h, wo_ref),
        ]
        cps = []
        for i, (h, v) in enumerate(pairs):
            cp = pltpu.make_async_copy(h, v, load_sems.at[i])
            cp.start()
            cps.append(cp)
        cp_wq, cp_wqr, cp_wo = cps[6], cps[7], cps[8]

        cps[0].wait()
        cps[1].wait()
        wdkrt = wsmall_ref[0:96].astype(BF)
        xs = []
        krs = []
        for b in range(B):
            cps[2 + b].wait()
            xs.append(x_ref[b].astype(BF))
            ckr = _dot_nt(xs[b], wdkrt)
            c_comm[0, b] = ckr[:, :DC_LOCAL].astype(BF)
            krs.append(ckr[:, DC_LOCAL:].astype(BF))
        cps[4].wait()
        wuk_comm[0] = wsmall_ref[96:160].astype(BF)
        cps[5].wait()
        wuv_comm[0] = wsmall_ref[160:224].astype(BF)

        barrier = pltpu.get_barrier_semaphore()
        pl.semaphore_signal(
            barrier, inc=1, device_id=partner,
            device_id_type=pl.DeviceIdType.MESH,
        )
        pl.semaphore_wait(barrier, 1)

        rdmas = []
        for i, buf in enumerate((c_comm, wuk_comm, wuv_comm)):
            rdma = pltpu.make_async_remote_copy(
                src_ref=buf.at[0],
                dst_ref=buf.at[1],
                send_sem=send_sems.at[i],
                recv_sem=recv_sems.at[i],
                device_id=partner,
                device_id_type=pl.DeviceIdType.MESH,
            )
            rdma.start()
            rdmas.append(rdma)

        cp_wq.wait()
        wq = (wq_ref[...] * SCALE).astype(BF)
        cp_wqr.wait()
        wqr = (wqr_ref[...] * SCALE).astype(BF)
        cp_wo.wait()
        wo = wo_ref[...].astype(BF)
        z32 = jnp.zeros((S, Dr), BF)
        one32 = jnp.ones((S, Dr), BF)
        for b in range(B):
            q_f = _dot(xs[b], wq)
            qr_f = _dot(xs[b], wqr)
            for h in range(H):
                c0 = 128 * h
                qcat[b, :, c0:c0 + 64] = q_f[:, 64 * h:64 * h + 64].astype(BF)
                qcat[b, :, c0 + 64:c0 + 96] = (
                    qr_f[:, 32 * h:32 * h + 32].astype(BF)
                )
                qcat[b, :, c0 + 96:c0 + 128] = z32

        for rdma in rdmas:
            rdma.wait()

        wuk_full = wuk_comm[...].reshape(DC, D)
        wuv_full = wuv_comm[...].reshape(DC, D)
        for b in range(B):
            ccat = jnp.concatenate([c_comm[0, b], c_comm[1, b]], axis=1)
            k_f = _dot(ccat, wuk_full)
            v_f = _dot(ccat, wuv_full)
            for h in range(H):
                c0 = 128 * h
                kcat[b, :, c0:c0 + 64] = k_f[:, 64 * h:64 * h + 64].astype(BF)
                kcat[b, :, c0 + 64:c0 + 96] = krs[b]
                kcat[b, :, c0 + 96:c0 + 128] = z32
                vcat[b, :, c0:c0 + 64] = v_f[:, 64 * h:64 * h + 64].astype(BF)
                vcat[b, :, c0 + 64:c0 + 96] = one32
                vcat[b, :, c0 + 96:c0 + 128] = z32
            o_heads = []
            for h in range(H):
                c0 = 128 * h
                e = jnp.exp(
                    _dot_nt(qcat[b, :, c0:c0 + 128], kcat[b, :, c0:c0 + 128])
                )
                o128 = _dot(e.astype(BF), vcat[b, :, c0:c0 + 128])
                o_heads.append(
                    (o128[:, :Dh] / o128[:, Dh:Dh + 1]).astype(BF)
                )
            out_ref[b] = _dot(jnp.concatenate(o_heads, axis=1), wo)

    hbm = lambda v: pltpu.with_memory_space_constraint(
        v, pltpu.MemorySpace.HBM
    )
    return pl.pallas_call(
        body,
        out_shape=jax.ShapeDtypeStruct((B, S, D), jnp.float32),
        in_specs=[pl.BlockSpec(memory_space=pltpu.MemorySpace.HBM)] * 8,
        out_specs=pl.BlockSpec(memory_space=pltpu.VMEM),
        scratch_shapes=[
            pltpu.VMEM((B, S, D), F32),
            pltpu.VMEM((224, D), F32),
            pltpu.VMEM((D, D), F32),
            pltpu.VMEM((D, H * Dr), F32),
            pltpu.VMEM((D, D), F32),
            pltpu.VMEM((2, B, S, DC_LOCAL), BF),
            pltpu.VMEM((2, DC_LOCAL, D), BF),
            pltpu.VMEM((2, DC_LOCAL, D), BF),
            pltpu.VMEM((B, S, H * 128), BF),
            pltpu.VMEM((B, S, H * 128), BF),
            pltpu.VMEM((B, S, H * 128), BF),
            pltpu.SemaphoreType.DMA((9,)),
            pltpu.SemaphoreType.DMA((3,)),
            pltpu.SemaphoreType.DMA((3,)),
        ],
        compiler_params=pltpu.CompilerParams(collective_id=0),
    )(
        hbm(x), hbm(Wdkv.T), hbm(Wkr.T), hbm(Wuk), hbm(Wuv),
        hbm(Wq), hbm(Wqr), hbm(Wo),
    )
